# baseline (speedup 1.0000x reference)
import jax
import jax.numpy as jnp
from jax.experimental import pallas as pl
from jax.experimental.pallas import tpu as pltpu

N = 100000
E = 1600000
H = 32


def _layer_xla(h, pos, src, dst, w1, b1, g, be, w2, b2):
    F = h.shape[1]
    A = h @ w1[:F] + pos @ w1[F:]
    C = -(pos @ w1[F:]) + b1
    x = A[src] + C[dst]
    mean = jnp.mean(x, axis=0)
    var = jnp.var(x, axis=0)
    y = jax.nn.relu((x - mean) / jnp.sqrt(var + 1e-5) * g + be)
    z = y @ w2 + b2
    m = jax.ops.segment_max(z, dst, num_segments=N)
    m = jnp.where(jnp.isneginf(m), 0.0, m)
    return m


def _cls_kernel(gh_ref, ph_ref, w_ref, b_ref, o_ref):
    feat = jnp.concatenate([gh_ref[...], ph_ref[...]], axis=-1)
    o_ref[...] = feat @ w_ref[...] + b_ref[...][None, :]


def kernel(pos, ph, edge_index, batch,
           c1_w1, c1_b1, c1_g, c1_be, c1_w2, c1_b2,
           c2_w1, c2_b1, c2_g, c2_be, c2_w2, c2_b2,
           cls_w, cls_b):
    src = edge_index[0]
    dst = edge_index[1]
    h = _layer_xla(pos, pos, src, dst, c1_w1, c1_b1, c1_g, c1_be, c1_w2, c1_b2)
    h = jax.nn.relu(h)
    h = _layer_xla(h, pos, src, dst, c2_w1, c2_b1, c2_g, c2_be, c2_w2, c2_b2)
    h = jax.nn.relu(h)
    g_h = jax.ops.segment_max(h, batch, num_segments=16)
    g_h = jnp.where(jnp.isneginf(g_h), 0.0, g_h)
    out = pl.pallas_call(
        _cls_kernel,
        out_shape=jax.ShapeDtypeStruct((16, 10), jnp.float32),
    )(g_h, ph, cls_w, cls_b)
    return out


# speedup vs baseline: 2.7929x; 2.7929x over previous
import functools
import jax
import jax.numpy as jnp
from jax import lax
from jax.experimental import pallas as pl
from jax.experimental.pallas import tpu as pltpu
from jax.experimental.pallas import tpu_sc as plsc

N = 100000
E = 1600000
H = 32
NW = 32
CHUNK = E // NW
B = 80
NB = CHUNK // B


def _sc_edge_body(a_hbm, c_hbm, src_hbm, dst_hbm, x_hbm, stats_hbm,
                  idx_s, idx_d, arows, crows, xblk, stat_v, sem_a, sem_c):
    wid = lax.axis_index("s") * 2 + lax.axis_index("c")
    zero = jnp.zeros((16,), jnp.float32)

    def blk(i, carry):
        s0, s1, q0, q1 = carry
        base = wid * CHUNK + i * B
        pltpu.sync_copy(src_hbm.at[pl.ds(base, B)], idx_s)
        pltpu.sync_copy(dst_hbm.at[pl.ds(base, B)], idx_d)
        cp_a = pltpu.async_copy(a_hbm.at[idx_s], arows, sem_a)
        cp_c = pltpu.async_copy(c_hbm.at[idx_d], crows, sem_c)
        cp_a.wait()
        cp_c.wait()

        def row(r, c2):
            t0, t1, u0, u1 = c2
            x0 = arows[r, pl.ds(0, 16)] + crows[r, pl.ds(0, 16)]
            x1 = arows[r, pl.ds(16, 16)] + crows[r, pl.ds(16, 16)]
            xblk[r, pl.ds(0, 16)] = x0
            xblk[r, pl.ds(16, 16)] = x1
            return (t0 + x0, t1 + x1, u0 + x0 * x0, u1 + x1 * x1)

        carry = lax.fori_loop(0, B, row, (s0, s1, q0, q1), unroll=8)
        pltpu.sync_copy(xblk, x_hbm.at[pl.ds(base, B)])
        return carry

    s0, s1, q0, q1 = lax.fori_loop(0, NB, blk, (zero, zero, zero, zero))
    stat_v[0, :] = s0
    stat_v[1, :] = s1
    stat_v[2, :] = q0
    stat_v[3, :] = q1
    pltpu.sync_copy(stat_v, stats_hbm.at[wid])


def _sc_edge_pass(a, c, src, dst):
    mesh = plsc.VectorSubcoreMesh(core_axis_name="c", subcore_axis_name="s")
    fn = pl.kernel(
        _sc_edge_body,
        out_type=(
            jax.ShapeDtypeStruct((E, H), jnp.float32),
            jax.ShapeDtypeStruct((NW, 4, 16), jnp.float32),
        ),
        mesh=mesh,
        compiler_params=pltpu.CompilerParams(use_tc_tiling_on_sc=False),
        scratch_types=(
            pltpu.VMEM((B,), jnp.int32),
            pltpu.VMEM((B,), jnp.int32),
            pltpu.VMEM((B, H), jnp.float32),
            pltpu.VMEM((B, H), jnp.float32),
            pltpu.VMEM((B, H), jnp.float32),
            pltpu.VMEM((4, 16), jnp.float32),
            pltpu.SemaphoreType.DMA,
            pltpu.SemaphoreType.DMA,
        ),
    )
    return fn(a, c, src, dst)


def _tables_body(h_ref, pos_ref, wh_ref, wp_ref, b1_ref, a_ref, c_ref):
    pw = jnp.dot(pos_ref[...], wp_ref[...], preferred_element_type=jnp.float32)
    a_ref[...] = jnp.dot(h_ref[...], wh_ref[...],
                         preferred_element_type=jnp.float32) + pw
    c_ref[...] = b1_ref[...] - pw


def _tables(h, pos, wh, wp, b1):
    nb = 50
    blk = N // nb
    f = h.shape[1]
    return pl.pallas_call(
        _tables_body,
        grid=(nb,),
        in_specs=[
            pl.BlockSpec((blk, f), lambda i: (i, 0)),
            pl.BlockSpec((blk, 3), lambda i: (i, 0)),
            pl.BlockSpec((f, H), lambda i: (0, 0)),
            pl.BlockSpec((3, H), lambda i: (0, 0)),
            pl.BlockSpec((1, H), lambda i: (0, 0)),
        ],
        out_specs=[
            pl.BlockSpec((blk, H), lambda i: (i, 0)),
            pl.BlockSpec((blk, H), lambda i: (i, 0)),
        ],
        out_shape=[
            jax.ShapeDtypeStruct((N, H), jnp.float32),
            jax.ShapeDtypeStruct((N, H), jnp.float32),
        ],
    )(h, pos, wh, wp, b1.reshape(1, H))


EBLK = 8000
ENB = E // EBLK


def _mlp1_body(x_ref, s_ref, t_ref, w2_ref, b2_ref, r_ref):
    y = jnp.maximum(x_ref[...] * s_ref[...] + t_ref[...], 0.0)
    z = jnp.dot(y, w2_ref[...], preferred_element_type=jnp.float32) + b2_ref[...]
    r_ref[...] = jnp.maximum(z, 0.0)


def _mlp1(x, s, t, w2, b2):
    return pl.pallas_call(
        _mlp1_body,
        grid=(ENB,),
        in_specs=[
            pl.BlockSpec((EBLK, H), lambda i: (i, 0)),
            pl.BlockSpec((1, H), lambda i: (0, 0)),
            pl.BlockSpec((1, H), lambda i: (0, 0)),
            pl.BlockSpec((H, H), lambda i: (0, 0)),
            pl.BlockSpec((1, H), lambda i: (0, 0)),
        ],
        out_specs=pl.BlockSpec((EBLK, H), lambda i: (i, 0)),
        out_shape=jax.ShapeDtypeStruct((E, H), jnp.float32),
    )(x, s.reshape(1, H), t.reshape(1, H), w2, b2.reshape(1, H))


def _mlp2_body(x_ref, dst_ref, bnd_ref, s_ref, t_ref, w2_ref, b2_ref, o_ref):
    @pl.when(pl.program_id(0) == 0)
    def _():
        o_ref[...] = jnp.zeros_like(o_ref)

    y = jnp.maximum(x_ref[...] * s_ref[...] + t_ref[...], 0.0)
    zt = lax.dot_general(w2_ref[...], y, (((0,), (1,)), ((), ())),
                         preferred_element_type=jnp.float32)
    rt = jnp.maximum(zt + b2_ref[...], 0.0)
    d = dst_ref[0]
    gid = jnp.zeros(d.shape, jnp.int32)
    for g in range(1, 16):
        gid = gid + jnp.where(d >= bnd_ref[g], 1, 0)
    for g in range(16):
        m = gid == g
        cmax = jnp.max(jnp.where(m, rt, 0.0), axis=1, keepdims=True)
        o_ref[:, pl.ds(g, 1)] = jnp.maximum(o_ref[:, pl.ds(g, 1)], cmax)


def _mlp2_pool(x, dst3d, bnd, s, t, w2, b2):
    return pl.pallas_call(
        _mlp2_body,
        grid=(ENB,),
        in_specs=[
            pl.BlockSpec((EBLK, H), lambda i: (i, 0)),
            pl.BlockSpec((1, 1, EBLK), lambda i: (i, 0, 0)),
            pl.BlockSpec(memory_space=pltpu.SMEM),
            pl.BlockSpec((1, H), lambda i: (0, 0)),
            pl.BlockSpec((1, H), lambda i: (0, 0)),
            pl.BlockSpec((H, H), lambda i: (0, 0)),
            pl.BlockSpec((H, 1), lambda i: (0, 0)),
        ],
        out_specs=pl.BlockSpec((H, 16), lambda i: (0, 0)),
        out_shape=jax.ShapeDtypeStruct((H, 16), jnp.float32),
    )(x, dst3d, bnd, s.reshape(1, H), t.reshape(1, H), w2, b2.reshape(H, 1))


def _cls_body(gh_ref, ph_ref, w_ref, b_ref, o_ref):
    feat = jnp.concatenate([gh_ref[...], ph_ref[...]], axis=-1)
    o_ref[...] = jnp.dot(feat, w_ref[...],
                         preferred_element_type=jnp.float32) + b_ref[...]


def _classifier(gh, ph, w, b):
    return pl.pallas_call(
        _cls_body,
        out_shape=jax.ShapeDtypeStruct((16, 10), jnp.float32),
    )(gh, ph, w, b.reshape(1, 10))


def _bn_fold(stats, g, be):
    sums = jnp.concatenate([stats[:, 0].sum(0), stats[:, 1].sum(0)])
    sqs = jnp.concatenate([stats[:, 2].sum(0), stats[:, 3].sum(0)])
    mean = sums / E
    var = sqs / E - mean * mean
    s = g * lax.rsqrt(var + 1e-5)
    t = be - mean * s
    return s, t


def kernel(pos, ph, edge_index, batch,
           c1_w1, c1_b1, c1_g, c1_be, c1_w2, c1_b2,
           c2_w1, c2_b1, c2_g, c2_be, c2_w2, c2_b2,
           cls_w, cls_b):
    src = edge_index[0]
    dst = edge_index[1]

    a1, c1t = _tables(pos, pos, c1_w1[:3], c1_w1[3:6], c1_b1)
    x1, st1 = _sc_edge_pass(a1, c1t, src, dst)
    s1, t1 = _bn_fold(st1, c1_g, c1_be)
    r1 = _mlp1(x1, s1, t1, c1_w2, c1_b2)
    h = jax.ops.segment_max(r1, dst, num_segments=N)
    h = jnp.where(jnp.isneginf(h), 0.0, h)

    a2, c2t = _tables(h, pos, c2_w1[:H], c2_w1[H:H + 3], c2_b1)
    x2, st2 = _sc_edge_pass(a2, c2t, src, dst)
    s2, t2 = _bn_fold(st2, c2_g, c2_be)
    bnd = jnp.searchsorted(batch, jnp.arange(16, dtype=jnp.int32)).astype(jnp.int32)
    dst3d = dst.reshape(ENB, 1, EBLK)
    ght = _mlp2_pool(x2, dst3d, bnd, s2, t2, c2_w2, c2_b2)

    return _classifier(ght.T, ph, cls_w, cls_b)
